# Initial kernel scaffold; baseline (speedup 1.0000x reference)
import jax
import jax.numpy as jnp
from jax.experimental import pallas as pl


def kernel(g_in, g_adj, batch_size, kernel_embeddings, reg_solutions, params):
    raise NotImplementedError("write your pallas kernel here")


# speedup vs baseline: 1.3889x; 1.3889x over previous
import jax
import jax.numpy as jnp
from jax.experimental import pallas as pl
from jax.experimental.pallas import tpu as pltpu

_H = 200


def _gru(x, h, wiT, whT, bi, bh):
    gx = jnp.dot(x, wiT, preferred_element_type=jnp.float32) + bi
    gh = jnp.dot(h, whT, preferred_element_type=jnp.float32) + bh
    r = jax.nn.sigmoid(gx[:, :_H] + gh[:, :_H])
    z = jax.nn.sigmoid(gx[:, _H:2 * _H] + gh[:, _H:2 * _H])
    n = jnp.tanh(gx[:, 2 * _H:] + r * gh[:, 2 * _H:])
    return (1.0 - z) * n + z * h


def _encode_kernel(adj_ref, gin_ref, ke_ref,
                   f0wi, f0wh, f0bi, f0bh,
                   f1wi, f1wh, f1bi, f1bh,
                   f2wi, f2wh, f2bi, f2bh,
                   b0wi, b0wh, b0bi, b0bh,
                   b1wi, b1wh, b1bi, b1bh,
                   vwi, vwh, vbi, vbh,
                   wmT_ref, bm_ref,
                   out_ref):
    b = pl.program_id(0)
    n = adj_ref.shape[1]

    adj = adj_ref[0]
    A = (adj < (16.0 / n)).astype(jnp.float32)
    AT = A.T
    deg_in = jnp.clip(jnp.sum(A, axis=1, keepdims=True), 1.0, None)
    deg_out = jnp.clip(jnp.sum(AT, axis=1, keepdims=True), 1.0, None)
    inv_in = 1.0 / deg_in
    inv_out = 1.0 / deg_out

    h = jnp.zeros((n, _H), dtype=jnp.float32)

    m = jnp.dot(A, gin_ref[0], preferred_element_type=jnp.float32) * inv_in
    h = _gru(m, h, f0wi[...], f0wh[...], f0bi[...], f0bh[...])
    out0 = h[0:8]
    mb = jnp.dot(AT, h, preferred_element_type=jnp.float32) * inv_out
    h = _gru(mb, h, b0wi[...], b0wh[...], b0bi[...], b0bh[...])

    m = jnp.dot(A, h, preferred_element_type=jnp.float32) * inv_in
    h = _gru(m, h, f1wi[...], f1wh[...], f1bi[...], f1bh[...])
    out1 = h[0:8]
    mb = jnp.dot(AT, h, preferred_element_type=jnp.float32) * inv_out
    h = _gru(mb, h, b1wi[...], b1wh[...], b1bi[...], b1bh[...])

    m = jnp.dot(A, h, preferred_element_type=jnp.float32) * inv_in
    h = _gru(m, h, f2wi[...], f2wh[...], f2bi[...], f2bh[...])
    out2 = h[0:8]

    hv = jnp.zeros((8, _H), dtype=jnp.float32)
    hv = _gru(out0, hv, vwi[...], vwh[...], vbi[...], vbh[...])
    hv = _gru(out1, hv, vwi[...], vwh[...], vbi[...], vbh[...])
    hv = _gru(out2, hv, vwi[...], vwh[...], vbi[...], vbh[...])

    wmT = wmT_ref[...]
    enc = (jnp.dot(hv[0:1], wmT[0:_H], preferred_element_type=jnp.float32)
           + jnp.dot(hv[1:2], wmT[_H:2 * _H], preferred_element_type=jnp.float32)
           + jnp.dot(hv[2:3], wmT[2 * _H:], preferred_element_type=jnp.float32)
           + bm_ref[...])
    diff = enc - ke_ref[pl.ds(b, 1), :]
    partial = jnp.sum(diff * diff)

    @pl.when(b == 0)
    def _():
        out_ref[...] = jnp.zeros_like(out_ref)

    out_ref[...] += partial


def kernel(g_in, g_adj, batch_size, kernel_embeddings, reg_solutions, params):
    B, N, VT = g_in.shape
    Z = kernel_embeddings.shape[1]

    def prep(p):
        return (p["Wi"].T, p["Wh"].T,
                p["bi"].reshape(1, -1), p["bh"].reshape(1, -1))

    wsets = (prep(params["fw"][0]) + prep(params["fw"][1]) +
             prep(params["fw"][2]) + prep(params["bw"][0]) +
             prep(params["bw"][1]) + prep(params["var"]))
    wmT = params["Wm"].T
    bm = params["bm"].reshape(1, -1)

    const = lambda shape: pl.BlockSpec(shape, lambda b: (0,) * len(shape))
    in_specs = [
        pl.BlockSpec((1, N, N), lambda b: (b, 0, 0)),
        pl.BlockSpec((1, N, VT), lambda b: (b, 0, 0)),
        const(kernel_embeddings.shape),
    ]
    in_specs += [const(w.shape) for w in wsets]
    in_specs += [const(wmT.shape), const(bm.shape)]

    loss = pl.pallas_call(
        _encode_kernel,
        grid=(B,),
        in_specs=in_specs,
        out_specs=pl.BlockSpec((1, 1), lambda b: (0, 0)),
        out_shape=jax.ShapeDtypeStruct((1, 1), jnp.float32),
        compiler_params=pltpu.CompilerParams(
            dimension_semantics=("arbitrary",),
        ),
    )(g_adj, g_in, kernel_embeddings, *wsets, wmT, bm)
    return loss[0, 0]


# speedup vs baseline: 1.7326x; 1.2474x over previous
import jax
import jax.numpy as jnp
from jax.experimental import pallas as pl
from jax.experimental.pallas import tpu as pltpu

_H = 200


def _gru(x, h, ws):
    wir, wiz, win, whr, whz, whn, br, bz, bni, bnh = ws
    dot = lambda a, b: jnp.dot(a, b, preferred_element_type=jnp.float32)
    r = jax.nn.sigmoid(dot(x, wir) + dot(h, whr) + br)
    z = jax.nn.sigmoid(dot(x, wiz) + dot(h, whz) + bz)
    n = jnp.tanh(dot(x, win) + bni + r * (dot(h, whn) + bnh))
    return (1.0 - z) * n + z * h


def _encode_kernel(*refs):
    adj_ref, gin_ref, ke_ref = refs[0], refs[1], refs[2]
    wsets = [tuple(r[...] for r in refs[3 + 10 * s: 3 + 10 * (s + 1)])
             for s in range(6)]
    fw0, fw1, fw2, bw0, bw1, var = wsets
    wmT_ref, bm_ref = refs[63], refs[64]
    out_ref = refs[65]

    b = pl.program_id(0)
    n = adj_ref.shape[1]

    adj = adj_ref[0]
    A = (adj < (16.0 / n)).astype(jnp.float32)
    AT = A.T
    inv_in = 1.0 / jnp.clip(jnp.sum(A, axis=1, keepdims=True), 1.0, None)
    inv_out = 1.0 / jnp.clip(jnp.sum(AT, axis=1, keepdims=True), 1.0, None)

    dot = lambda a, b: jnp.dot(a, b, preferred_element_type=jnp.float32)
    h = jnp.zeros((n, _H), dtype=jnp.float32)

    h = _gru(dot(A, gin_ref[0]) * inv_in, h, fw0)
    out0 = h[0:8]
    h = _gru(dot(AT, h) * inv_out, h, bw0)
    h = _gru(dot(A, h) * inv_in, h, fw1)
    out1 = h[0:8]
    h = _gru(dot(AT, h) * inv_out, h, bw1)
    h = _gru(dot(A, h) * inv_in, h, fw2)
    out2 = h[0:8]

    hv = jnp.zeros((8, _H), dtype=jnp.float32)
    hv = _gru(out0, hv, var)
    hv = _gru(out1, hv, var)
    hv = _gru(out2, hv, var)

    wmT = wmT_ref[...]
    enc = (dot(hv[0:1], wmT[0:_H])
           + dot(hv[1:2], wmT[_H:2 * _H])
           + dot(hv[2:3], wmT[2 * _H:])
           + bm_ref[...])
    diff = enc - ke_ref[pl.ds(b, 1), :]
    partial = jnp.sum(diff * diff)

    @pl.when(b == 0)
    def _():
        out_ref[...] = jnp.zeros_like(out_ref)

    out_ref[...] += partial


def kernel(g_in, g_adj, batch_size, kernel_embeddings, reg_solutions, params):
    B, N, VT = g_in.shape

    def prep(p):
        wiT = p["Wi"].T
        whT = p["Wh"].T
        bi, bh = p["bi"], p["bh"]
        return (wiT[:, :_H], wiT[:, _H:2 * _H], wiT[:, 2 * _H:],
                whT[:, :_H], whT[:, _H:2 * _H], whT[:, 2 * _H:],
                (bi[:_H] + bh[:_H]).reshape(1, _H),
                (bi[_H:2 * _H] + bh[_H:2 * _H]).reshape(1, _H),
                bi[2 * _H:].reshape(1, _H),
                bh[2 * _H:].reshape(1, _H))

    wsets = (prep(params["fw"][0]) + prep(params["fw"][1]) +
             prep(params["fw"][2]) + prep(params["bw"][0]) +
             prep(params["bw"][1]) + prep(params["var"]))
    wmT = params["Wm"].T
    bm = params["bm"].reshape(1, -1)

    const = lambda shape: pl.BlockSpec(shape, lambda b: (0,) * len(shape))
    in_specs = [
        pl.BlockSpec((1, N, N), lambda b: (b, 0, 0)),
        pl.BlockSpec((1, N, VT), lambda b: (b, 0, 0)),
        const(kernel_embeddings.shape),
    ]
    in_specs += [const(w.shape) for w in wsets]
    in_specs += [const(wmT.shape), const(bm.shape)]

    loss = pl.pallas_call(
        _encode_kernel,
        grid=(B,),
        in_specs=in_specs,
        out_specs=pl.BlockSpec((1, 1), lambda b: (0, 0)),
        out_shape=jax.ShapeDtypeStruct((1, 1), jnp.float32),
        compiler_params=pltpu.CompilerParams(
            dimension_semantics=("arbitrary",),
        ),
    )(g_adj, g_in, kernel_embeddings, *wsets, wmT, bm)
    return loss[0, 0]


# speedup vs baseline: 1.7820x; 1.0285x over previous
import jax
import jax.numpy as jnp
from jax.experimental import pallas as pl
from jax.experimental.pallas import tpu as pltpu

_H = 200


def _gru(x, h, ws):
    wir, wiz, win, whr, whz, whn, br, bz, bni, bnh = ws
    dot = lambda a, b: jnp.dot(a, b, preferred_element_type=jnp.float32)
    xb = x.astype(jnp.bfloat16)
    hb = h.astype(jnp.bfloat16)
    r = jax.nn.sigmoid(dot(xb, wir) + dot(hb, whr) + br)
    z = jax.nn.sigmoid(dot(xb, wiz) + dot(hb, whz) + bz)
    n = jnp.tanh(dot(xb, win) + bni + r * (dot(hb, whn) + bnh))
    return (1.0 - z) * n + z * h


def _encode_kernel(*refs):
    adj_ref, gin_ref, ke_ref = refs[0], refs[1], refs[2]
    wsets = [tuple(r[...] for r in refs[3 + 10 * s: 3 + 10 * (s + 1)])
             for s in range(6)]
    fw0, fw1, fw2, bw0, bw1, var = wsets
    wmT_ref, bm_ref = refs[63], refs[64]
    out_ref = refs[65]

    b = pl.program_id(0)
    n = adj_ref.shape[1]

    adj = adj_ref[0]
    Af = (adj < (16.0 / n)).astype(jnp.float32)
    A = Af.astype(jnp.bfloat16)
    AT = A.T
    inv_in = 1.0 / jnp.clip(jnp.sum(Af, axis=1, keepdims=True), 1.0, None)
    inv_out = 1.0 / jnp.clip(jnp.sum(Af, axis=0, keepdims=True), 1.0, None).T

    dot = lambda a, b: jnp.dot(a, b, preferred_element_type=jnp.float32)
    h = jnp.zeros((n, _H), dtype=jnp.float32)

    bf = lambda v: v.astype(jnp.bfloat16)
    h = _gru(dot(A, gin_ref[0]) * inv_in, h, fw0)
    out0 = h[0:8]
    h = _gru(dot(AT, bf(h)) * inv_out, h, bw0)
    h = _gru(dot(A, bf(h)) * inv_in, h, fw1)
    out1 = h[0:8]
    h = _gru(dot(AT, bf(h)) * inv_out, h, bw1)
    h = _gru(dot(A, bf(h)) * inv_in, h, fw2)
    out2 = h[0:8]

    hv = jnp.zeros((8, _H), dtype=jnp.float32)
    hv = _gru(out0, hv, var)
    hv = _gru(out1, hv, var)
    hv = _gru(out2, hv, var)

    wmT = wmT_ref[...]
    hvb = hv.astype(jnp.bfloat16)
    enc = (dot(hvb[0:1], wmT[0:_H])
           + dot(hvb[1:2], wmT[_H:2 * _H])
           + dot(hvb[2:3], wmT[2 * _H:])
           + bm_ref[...])
    diff = enc - ke_ref[pl.ds(b, 1), :]
    partial = jnp.sum(diff * diff)

    @pl.when(b == 0)
    def _():
        out_ref[...] = jnp.zeros_like(out_ref)

    out_ref[...] += partial


def kernel(g_in, g_adj, batch_size, kernel_embeddings, reg_solutions, params):
    B, N, VT = g_in.shape

    def prep(p):
        wiT = p["Wi"].T.astype(jnp.bfloat16)
        whT = p["Wh"].T.astype(jnp.bfloat16)
        bi, bh = p["bi"], p["bh"]
        return (wiT[:, :_H], wiT[:, _H:2 * _H], wiT[:, 2 * _H:],
                whT[:, :_H], whT[:, _H:2 * _H], whT[:, 2 * _H:],
                (bi[:_H] + bh[:_H]).reshape(1, _H),
                (bi[_H:2 * _H] + bh[_H:2 * _H]).reshape(1, _H),
                bi[2 * _H:].reshape(1, _H),
                bh[2 * _H:].reshape(1, _H))

    wsets = (prep(params["fw"][0]) + prep(params["fw"][1]) +
             prep(params["fw"][2]) + prep(params["bw"][0]) +
             prep(params["bw"][1]) + prep(params["var"]))
    wmT = params["Wm"].T.astype(jnp.bfloat16)
    bm = params["bm"].reshape(1, -1)

    const = lambda shape: pl.BlockSpec(shape, lambda b: (0,) * len(shape))
    in_specs = [
        pl.BlockSpec((1, N, N), lambda b: (b, 0, 0)),
        pl.BlockSpec((1, N, VT), lambda b: (b, 0, 0)),
        const(kernel_embeddings.shape),
    ]
    in_specs += [const(w.shape) for w in wsets]
    in_specs += [const(wmT.shape), const(bm.shape)]

    loss = pl.pallas_call(
        _encode_kernel,
        grid=(B,),
        in_specs=in_specs,
        out_specs=pl.BlockSpec((1, 1), lambda b: (0, 0)),
        out_shape=jax.ShapeDtypeStruct((1, 1), jnp.float32),
        compiler_params=pltpu.CompilerParams(
            dimension_semantics=("arbitrary",),
        ),
    )(g_adj, g_in.astype(jnp.bfloat16), kernel_embeddings, *wsets, wmT, bm)
    return loss[0, 0]


# speedup vs baseline: 2.0602x; 1.1561x over previous
import jax
import jax.numpy as jnp
from jax.experimental import pallas as pl
from jax.experimental.pallas import tpu as pltpu

_H = 200
_G = 4


def _dot(a, b):
    return jnp.dot(a, b, preferred_element_type=jnp.float32)


def _gru(x, h, ws):
    wir, wiz, win, whr, whz, whn, br, bz, bni, bnh = ws
    xb = x.astype(jnp.bfloat16)
    hb = h.astype(jnp.bfloat16)
    r = jax.nn.sigmoid(_dot(xb, wir) + _dot(hb, whr) + br)
    z = jax.nn.sigmoid(_dot(xb, wiz) + _dot(hb, whz) + bz)
    n = jnp.tanh(_dot(xb, win) + bni + r * (_dot(hb, whn) + bnh))
    return n + z * (h - n)


def _encode_kernel(*refs):
    adj_ref, gin_ref, ke_ref = refs[0], refs[1], refs[2]
    wsets = [tuple(r[...] for r in refs[3 + 10 * s: 3 + 10 * (s + 1)])
             for s in range(6)]
    fw0, fw1, fw2, bw0, bw1, var = wsets
    wmT_ref, bm_ref = refs[63], refs[64]
    out_ref = refs[65]

    step = pl.program_id(0)
    n = adj_ref.shape[2]

    An, ATn = [], []
    for g in range(_G):
        Af = (adj_ref[0, g] < (16.0 / n)).astype(jnp.float32)
        inv_in = 1.0 / jnp.clip(jnp.sum(Af, axis=1, keepdims=True), 1.0, None)
        inv_out = (1.0 / jnp.clip(jnp.sum(Af, axis=0, keepdims=True), 1.0,
                                  None)).T
        An.append((Af * inv_in).astype(jnp.bfloat16))
        ATn.append((Af.T * inv_out).astype(jnp.bfloat16))

    gin = gin_ref[0]

    def fwd_msg(feat):
        return jnp.concatenate(
            [_dot(An[g], feat[g * n:(g + 1) * n]) for g in range(_G)], axis=0)

    def bwd_msg(feat):
        return jnp.concatenate(
            [_dot(ATn[g], feat[g * n:(g + 1) * n]) for g in range(_G)], axis=0)

    bf = lambda v: v.astype(jnp.bfloat16)
    h = jnp.zeros((_G * n, _H), dtype=jnp.float32)

    def snap(hcur):
        rows = [hcur[g * n + k: g * n + k + 1]
                for k in range(3) for g in range(_G)]
        rows.append(jnp.zeros((16 - 3 * _G, _H), dtype=jnp.float32))
        return jnp.concatenate(rows, axis=0)

    h = _gru(fwd_msg(gin), h, fw0)
    out0 = snap(h)
    h = _gru(bwd_msg(bf(h)), h, bw0)
    h = _gru(fwd_msg(bf(h)), h, fw1)
    out1 = snap(h)
    h = _gru(bwd_msg(bf(h)), h, bw1)
    h = _gru(fwd_msg(bf(h)), h, fw2)
    out2 = snap(h)

    hv = jnp.zeros((16, _H), dtype=jnp.float32)
    hv = _gru(out0, hv, var)
    hv = _gru(out1, hv, var)
    hv = _gru(out2, hv, var)

    wmT = wmT_ref[...]
    hvb = hv.astype(jnp.bfloat16)
    enc = (_dot(hvb[0:_G], wmT[0:_H])
           + _dot(hvb[_G:2 * _G], wmT[_H:2 * _H])
           + _dot(hvb[2 * _G:3 * _G], wmT[2 * _H:])
           + bm_ref[...])
    diff = enc - ke_ref[0]
    partial = jnp.sum(diff * diff)

    @pl.when(step == 0)
    def _():
        out_ref[...] = jnp.zeros_like(out_ref)

    out_ref[...] += partial


def kernel(g_in, g_adj, batch_size, kernel_embeddings, reg_solutions, params):
    B, N, VT = g_in.shape
    Z = kernel_embeddings.shape[1]
    steps = B // _G

    def prep(p):
        wiT = p["Wi"].T.astype(jnp.bfloat16)
        whT = p["Wh"].T.astype(jnp.bfloat16)
        bi, bh = p["bi"], p["bh"]
        return (wiT[:, :_H], wiT[:, _H:2 * _H], wiT[:, 2 * _H:],
                whT[:, :_H], whT[:, _H:2 * _H], whT[:, 2 * _H:],
                (bi[:_H] + bh[:_H]).reshape(1, _H),
                (bi[_H:2 * _H] + bh[_H:2 * _H]).reshape(1, _H),
                bi[2 * _H:].reshape(1, _H),
                bh[2 * _H:].reshape(1, _H))

    wsets = (prep(params["fw"][0]) + prep(params["fw"][1]) +
             prep(params["fw"][2]) + prep(params["bw"][0]) +
             prep(params["bw"][1]) + prep(params["var"]))
    wmT = params["Wm"].T.astype(jnp.bfloat16)
    bm = params["bm"].reshape(1, -1)

    adj4 = g_adj.reshape(steps, _G, N, N)
    gin2 = g_in.astype(jnp.bfloat16).reshape(steps, _G * N, VT)
    ke3 = kernel_embeddings.reshape(steps, _G, Z)

    const = lambda shape: pl.BlockSpec(shape, lambda s: (0,) * len(shape))
    in_specs = [
        pl.BlockSpec((1, _G, N, N), lambda s: (s, 0, 0, 0)),
        pl.BlockSpec((1, _G * N, VT), lambda s: (s, 0, 0)),
        pl.BlockSpec((1, _G, Z), lambda s: (s, 0, 0)),
    ]
    in_specs += [const(w.shape) for w in wsets]
    in_specs += [const(wmT.shape), const(bm.shape)]

    loss = pl.pallas_call(
        _encode_kernel,
        grid=(steps,),
        in_specs=in_specs,
        out_specs=pl.BlockSpec((1, 1), lambda s: (0, 0)),
        out_shape=jax.ShapeDtypeStruct((1, 1), jnp.float32),
        compiler_params=pltpu.CompilerParams(
            dimension_semantics=("arbitrary",),
        ),
    )(adj4, gin2, ke3, *wsets, wmT, bm)
    return loss[0, 0]


# speedup vs baseline: 2.3289x; 1.1304x over previous
import jax
import jax.numpy as jnp
from jax.experimental import pallas as pl
from jax.experimental.pallas import tpu as pltpu

_H = 200
_G = 4


def _dot(a, b):
    return jnp.dot(a, b, preferred_element_type=jnp.float32)


def _gru(x, h, ws):
    wir, wiz, win, whr, whz, whn, br, bz, bni, bnh = ws
    xb = x.astype(jnp.bfloat16)
    hb = h.astype(jnp.bfloat16)
    r = jax.nn.sigmoid(_dot(xb, wir) + _dot(hb, whr) + br)
    z = jax.nn.sigmoid(_dot(xb, wiz) + _dot(hb, whz) + bz)
    n = jnp.tanh(_dot(xb, win) + bni + r * (_dot(hb, whn) + bnh))
    return n + z * (h - n)


def _encode_kernel(adj_ref, gin_ref, ke_ref, wi0_ref, wi5_ref, wh6_ref,
                   bi_ref, bh_ref, wmT_ref, bm_ref, out_ref):
    step = pl.program_id(0)
    n = adj_ref.shape[2]

    def wset(s):
        if s == 0:
            wi = (wi0_ref[0], wi0_ref[1], wi0_ref[2])
        else:
            wi = (wi5_ref[s - 1, 0], wi5_ref[s - 1, 1], wi5_ref[s - 1, 2])
        wh = (wh6_ref[s, 0], wh6_ref[s, 1], wh6_ref[s, 2])
        br = bi_ref[3 * s:3 * s + 1] + bh_ref[3 * s:3 * s + 1]
        bz = bi_ref[3 * s + 1:3 * s + 2] + bh_ref[3 * s + 1:3 * s + 2]
        bni = bi_ref[3 * s + 2:3 * s + 3]
        bnh = bh_ref[3 * s + 2:3 * s + 3]
        return wi + wh + (br, bz, bni, bnh)

    fw0, fw1, fw2, bw0, bw1, var = [wset(s) for s in range(6)]

    An, ATn = [], []
    for g in range(_G):
        Af = (adj_ref[0, g] < (16.0 / n)).astype(jnp.float32)
        inv_in = 1.0 / jnp.clip(jnp.sum(Af, axis=1, keepdims=True), 1.0, None)
        inv_out = (1.0 / jnp.clip(jnp.sum(Af, axis=0, keepdims=True), 1.0,
                                  None)).T
        An.append((Af * inv_in).astype(jnp.bfloat16))
        ATn.append((Af.T * inv_out).astype(jnp.bfloat16))

    gin = gin_ref[0]

    def fwd_msg(feat):
        return jnp.concatenate(
            [_dot(An[g], feat[g * n:(g + 1) * n]) for g in range(_G)], axis=0)

    def bwd_msg(feat):
        return jnp.concatenate(
            [_dot(ATn[g], feat[g * n:(g + 1) * n]) for g in range(_G)], axis=0)

    bf = lambda v: v.astype(jnp.bfloat16)
    h = jnp.zeros((_G * n, _H), dtype=jnp.float32)

    def snap(hcur):
        rows = [hcur[g * n + k: g * n + k + 1]
                for k in range(3) for g in range(_G)]
        rows.append(jnp.zeros((16 - 3 * _G, _H), dtype=jnp.float32))
        return jnp.concatenate(rows, axis=0)

    h = _gru(fwd_msg(gin), h, fw0)
    out0 = snap(h)
    h = _gru(bwd_msg(bf(h)), h, bw0)
    h = _gru(fwd_msg(bf(h)), h, fw1)
    out1 = snap(h)
    h = _gru(bwd_msg(bf(h)), h, bw1)
    h = _gru(fwd_msg(bf(h)), h, fw2)
    out2 = snap(h)

    hv = jnp.zeros((16, _H), dtype=jnp.float32)
    hv = _gru(out0, hv, var)
    hv = _gru(out1, hv, var)
    hv = _gru(out2, hv, var)

    wmT = wmT_ref[...]
    hvb = hv.astype(jnp.bfloat16)
    enc = (_dot(hvb[0:_G], wmT[0:_H])
           + _dot(hvb[_G:2 * _G], wmT[_H:2 * _H])
           + _dot(hvb[2 * _G:3 * _G], wmT[2 * _H:])
           + bm_ref[...])
    diff = enc - ke_ref[0]
    partial = jnp.sum(diff * diff)

    @pl.when(step == 0)
    def _():
        out_ref[...] = jnp.zeros_like(out_ref)

    out_ref[...] += partial


def kernel(g_in, g_adj, batch_size, kernel_embeddings, reg_solutions, params):
    B, N, VT = g_in.shape
    Z = kernel_embeddings.shape[1]
    steps = B // _G
    p = params

    wi0 = (p["fw"][0]["Wi"].reshape(3, _H, VT).transpose(0, 2, 1)
           .astype(jnp.bfloat16))
    wi5 = (jnp.stack([p["fw"][1]["Wi"], p["fw"][2]["Wi"], p["bw"][0]["Wi"],
                      p["bw"][1]["Wi"], p["var"]["Wi"]])
           .reshape(5, 3, _H, _H).transpose(0, 1, 3, 2).astype(jnp.bfloat16))
    wh6 = (jnp.stack([p["fw"][0]["Wh"], p["fw"][1]["Wh"], p["fw"][2]["Wh"],
                      p["bw"][0]["Wh"], p["bw"][1]["Wh"], p["var"]["Wh"]])
           .reshape(6, 3, _H, _H).transpose(0, 1, 3, 2).astype(jnp.bfloat16))
    bi18 = jnp.stack([p["fw"][0]["bi"], p["fw"][1]["bi"], p["fw"][2]["bi"],
                      p["bw"][0]["bi"], p["bw"][1]["bi"], p["var"]["bi"]]
                     ).reshape(18, _H)
    bh18 = jnp.stack([p["fw"][0]["bh"], p["fw"][1]["bh"], p["fw"][2]["bh"],
                      p["bw"][0]["bh"], p["bw"][1]["bh"], p["var"]["bh"]]
                     ).reshape(18, _H)
    wmT = p["Wm"].T.astype(jnp.bfloat16)
    bm = p["bm"].reshape(1, -1)

    adj4 = g_adj.reshape(steps, _G, N, N)
    gin2 = g_in.astype(jnp.bfloat16).reshape(steps, _G * N, VT)
    ke3 = kernel_embeddings.reshape(steps, _G, Z)

    const = lambda shape: pl.BlockSpec(shape, lambda s: (0,) * len(shape))
    in_specs = [
        pl.BlockSpec((1, _G, N, N), lambda s: (s, 0, 0, 0)),
        pl.BlockSpec((1, _G * N, VT), lambda s: (s, 0, 0)),
        pl.BlockSpec((1, _G, Z), lambda s: (s, 0, 0)),
        const(wi0.shape), const(wi5.shape), const(wh6.shape),
        const(bi18.shape), const(bh18.shape), const(wmT.shape),
        const(bm.shape),
    ]

    loss = pl.pallas_call(
        _encode_kernel,
        grid=(steps,),
        in_specs=in_specs,
        out_specs=pl.BlockSpec((1, 1), lambda s: (0, 0)),
        out_shape=jax.ShapeDtypeStruct((1, 1), jnp.float32),
        compiler_params=pltpu.CompilerParams(
            dimension_semantics=("arbitrary",),
        ),
    )(adj4, gin2, ke3, wi0, wi5, wh6, bi18, bh18, wmT, bm)
    return loss[0, 0]


# speedup vs baseline: 2.4811x; 1.0654x over previous
import jax
import jax.numpy as jnp
from jax.experimental import pallas as pl
from jax.experimental.pallas import tpu as pltpu

_H = 200
_G = 4


def _dot(a, b):
    return jnp.dot(a, b, preferred_element_type=jnp.float32)


def _gru(x, h, ws):
    wir, wiz, win, whr, whz, whn, br, bz, bni, bnh = ws
    xb = x.astype(jnp.bfloat16)
    hb = h.astype(jnp.bfloat16)
    r = jax.nn.sigmoid(_dot(xb, wir) + _dot(hb, whr) + br)
    z = jax.nn.sigmoid(_dot(xb, wiz) + _dot(hb, whz) + bz)
    n = jnp.tanh(_dot(xb, win) + bni + r * (_dot(hb, whn) + bnh))
    return n + z * (h - n)


def _encode_kernel(adj_ref, gin_ref, ke_ref, wi0_ref, wi5_ref, wh6_ref,
                   bi_ref, bh_ref, wmT_ref, bm_ref, out_ref):
    step = pl.program_id(0)
    n = adj_ref.shape[2]

    def wset(s):
        if s == 0:
            wi = (wi0_ref[0], wi0_ref[1], wi0_ref[2])
        else:
            wi = (wi5_ref[s - 1, 0], wi5_ref[s - 1, 1], wi5_ref[s - 1, 2])
        wh = (wh6_ref[s, 0], wh6_ref[s, 1], wh6_ref[s, 2])
        br = bi_ref[3 * s:3 * s + 1] + bh_ref[3 * s:3 * s + 1]
        bz = bi_ref[3 * s + 1:3 * s + 2] + bh_ref[3 * s + 1:3 * s + 2]
        bni = bi_ref[3 * s + 2:3 * s + 3]
        bnh = bh_ref[3 * s + 2:3 * s + 3]
        return wi + wh + (br, bz, bni, bnh)

    fw0, fw1, fw2, bw0, bw1, var = [wset(s) for s in range(6)]

    An, degs, invouts = [], [], []
    for g in range(_G):
        Af = (adj_ref[0, g] < (16.0 / n)).astype(jnp.float32)
        deg_in = jnp.clip(jnp.sum(Af, axis=1, keepdims=True), 1.0, None)
        inv_out = (1.0 / jnp.clip(jnp.sum(Af, axis=0, keepdims=True), 1.0,
                                  None)).T
        An.append((Af * (1.0 / deg_in)).astype(jnp.bfloat16))
        degs.append(deg_in)
        invouts.append(inv_out)

    gin = gin_ref[0].astype(jnp.bfloat16)

    def fwd_msg(feat):
        return jnp.concatenate(
            [_dot(An[g], feat[g * n:(g + 1) * n]) for g in range(_G)], axis=0)

    def bwd_msg(h):
        outs = []
        for g in range(_G):
            hg = (h[g * n:(g + 1) * n] * degs[g]).astype(jnp.bfloat16)
            mg = jax.lax.dot_general(
                An[g], hg, (((0,), (0,)), ((), ())),
                preferred_element_type=jnp.float32)
            outs.append(mg * invouts[g])
        return jnp.concatenate(outs, axis=0)

    bf = lambda v: v.astype(jnp.bfloat16)
    h = jnp.zeros((_G * n, _H), dtype=jnp.float32)

    def snap(hcur):
        rows = [hcur[g * n + k: g * n + k + 1]
                for k in range(3) for g in range(_G)]
        rows.append(jnp.zeros((16 - 3 * _G, _H), dtype=jnp.float32))
        return jnp.concatenate(rows, axis=0)

    h = _gru(fwd_msg(gin), h, fw0)
    out0 = snap(h)
    h = _gru(bwd_msg(h), h, bw0)
    h = _gru(fwd_msg(bf(h)), h, fw1)
    out1 = snap(h)
    h = _gru(bwd_msg(h), h, bw1)
    h = _gru(fwd_msg(bf(h)), h, fw2)
    out2 = snap(h)

    hv = jnp.zeros((16, _H), dtype=jnp.float32)
    hv = _gru(out0, hv, var)
    hv = _gru(out1, hv, var)
    hv = _gru(out2, hv, var)

    wmT = wmT_ref[...]
    hvb = hv.astype(jnp.bfloat16)
    enc = (_dot(hvb[0:_G], wmT[0:_H])
           + _dot(hvb[_G:2 * _G], wmT[_H:2 * _H])
           + _dot(hvb[2 * _G:3 * _G], wmT[2 * _H:])
           + bm_ref[...])
    diff = enc - ke_ref[0]
    del step
    out_ref[0] = jnp.sum(diff * diff, keepdims=True).reshape(1, 1)


def kernel(g_in, g_adj, batch_size, kernel_embeddings, reg_solutions, params):
    B, N, VT = g_in.shape
    Z = kernel_embeddings.shape[1]
    steps = B // _G
    p = params

    wi0 = (p["fw"][0]["Wi"].reshape(3, _H, VT).transpose(0, 2, 1)
           .astype(jnp.bfloat16))
    wi5 = (jnp.stack([p["fw"][1]["Wi"], p["fw"][2]["Wi"], p["bw"][0]["Wi"],
                      p["bw"][1]["Wi"], p["var"]["Wi"]])
           .reshape(5, 3, _H, _H).transpose(0, 1, 3, 2).astype(jnp.bfloat16))
    wh6 = (jnp.stack([p["fw"][0]["Wh"], p["fw"][1]["Wh"], p["fw"][2]["Wh"],
                      p["bw"][0]["Wh"], p["bw"][1]["Wh"], p["var"]["Wh"]])
           .reshape(6, 3, _H, _H).transpose(0, 1, 3, 2).astype(jnp.bfloat16))
    bi18 = jnp.stack([p["fw"][0]["bi"], p["fw"][1]["bi"], p["fw"][2]["bi"],
                      p["bw"][0]["bi"], p["bw"][1]["bi"], p["var"]["bi"]]
                     ).reshape(18, _H)
    bh18 = jnp.stack([p["fw"][0]["bh"], p["fw"][1]["bh"], p["fw"][2]["bh"],
                      p["bw"][0]["bh"], p["bw"][1]["bh"], p["var"]["bh"]]
                     ).reshape(18, _H)
    wmT = p["Wm"].T.astype(jnp.bfloat16)
    bm = p["bm"].reshape(1, -1)

    adj4 = g_adj.reshape(steps, _G, N, N)
    gin2 = g_in.reshape(steps, _G * N, VT)
    ke3 = kernel_embeddings.reshape(steps, _G, Z)

    const = lambda shape: pl.BlockSpec(shape, lambda s: (0,) * len(shape))
    in_specs = [
        pl.BlockSpec((1, _G, N, N), lambda s: (s, 0, 0, 0)),
        pl.BlockSpec((1, _G * N, VT), lambda s: (s, 0, 0)),
        pl.BlockSpec((1, _G, Z), lambda s: (s, 0, 0)),
        const(wi0.shape), const(wi5.shape), const(wh6.shape),
        const(bi18.shape), const(bh18.shape), const(wmT.shape),
        const(bm.shape),
    ]

    loss = pl.pallas_call(
        _encode_kernel,
        grid=(steps,),
        in_specs=in_specs,
        out_specs=pl.BlockSpec((1, 1, 1), lambda s: (s, 0, 0)),
        out_shape=jax.ShapeDtypeStruct((steps, 1, 1), jnp.float32),
        compiler_params=pltpu.CompilerParams(
            dimension_semantics=("parallel",),
        ),
    )(adj4, gin2, ke3, wi0, wi5, wh6, bi18, bh18, wmT, bm)
    return jnp.sum(loss)


# speedup vs baseline: 2.4995x; 1.0074x over previous
import jax
import jax.numpy as jnp
from jax.experimental import pallas as pl
from jax.experimental.pallas import tpu as pltpu

_H = 200
_G = 4


def _dot(a, b):
    return jnp.dot(a, b, preferred_element_type=jnp.float32)


def _dot_nt(a, b):
    return jax.lax.dot_general(a, b, (((1,), (1,)), ((), ())),
                               preferred_element_type=jnp.float32)


def _gru(x, h, ws):
    wir, wiz, win, whr, whz, whn, br, bz, bni, bnh = ws
    xb = x.astype(jnp.bfloat16)
    hb = h.astype(jnp.bfloat16)
    r = jax.nn.sigmoid(_dot_nt(xb, wir) + _dot_nt(hb, whr) + br)
    z = jax.nn.sigmoid(_dot_nt(xb, wiz) + _dot_nt(hb, whz) + bz)
    n = jnp.tanh(_dot_nt(xb, win) + bni + r * (_dot_nt(hb, whn) + bnh))
    return n + z * (h - n)


def _encode_kernel(adj_ref, gin_ref, ke_ref, wi0_ref, wi5_ref, wh6_ref,
                   bi_ref, bh_ref, wm_ref, bm_ref, out_ref):
    step = pl.program_id(0)
    n = adj_ref.shape[2]

    def wset(s):
        if s == 0:
            wi = (wi0_ref[0], wi0_ref[1], wi0_ref[2])
        else:
            wi = (wi5_ref[s - 1, 0], wi5_ref[s - 1, 1], wi5_ref[s - 1, 2])
        wh = (wh6_ref[s, 0], wh6_ref[s, 1], wh6_ref[s, 2])
        br = bi_ref[3 * s:3 * s + 1] + bh_ref[3 * s:3 * s + 1]
        bz = bi_ref[3 * s + 1:3 * s + 2] + bh_ref[3 * s + 1:3 * s + 2]
        bni = bi_ref[3 * s + 2:3 * s + 3]
        bnh = bh_ref[3 * s + 2:3 * s + 3]
        return wi + wh + (br, bz, bni, bnh)

    fw0, fw1, fw2, bw0, bw1, var = [wset(s) for s in range(6)]

    An, degs, invouts = [], [], []
    for g in range(_G):
        Af = (adj_ref[0, g] < (16.0 / n)).astype(jnp.float32)
        deg_in = jnp.clip(jnp.sum(Af, axis=1, keepdims=True), 1.0, None)
        inv_out = (1.0 / jnp.clip(jnp.sum(Af, axis=0, keepdims=True), 1.0,
                                  None)).T
        An.append((Af * (1.0 / deg_in)).astype(jnp.bfloat16))
        degs.append(deg_in)
        invouts.append(inv_out)

    gin = gin_ref[0].astype(jnp.bfloat16)

    def fwd_msg(feat):
        return jnp.concatenate(
            [_dot(An[g], feat[g * n:(g + 1) * n]) for g in range(_G)], axis=0)

    def bwd_msg(h):
        outs = []
        for g in range(_G):
            hg = (h[g * n:(g + 1) * n] * degs[g]).astype(jnp.bfloat16)
            mg = jax.lax.dot_general(
                An[g], hg, (((0,), (0,)), ((), ())),
                preferred_element_type=jnp.float32)
            outs.append(mg * invouts[g])
        return jnp.concatenate(outs, axis=0)

    bf = lambda v: v.astype(jnp.bfloat16)
    h = jnp.zeros((_G * n, _H), dtype=jnp.float32)

    def snap(hcur):
        rows = [hcur[g * n + k: g * n + k + 1]
                for k in range(3) for g in range(_G)]
        rows.append(jnp.zeros((16 - 3 * _G, _H), dtype=jnp.float32))
        return jnp.concatenate(rows, axis=0)

    h = _gru(fwd_msg(gin), h, fw0)
    out0 = snap(h)
    h = _gru(bwd_msg(h), h, bw0)
    h = _gru(fwd_msg(bf(h)), h, fw1)
    out1 = snap(h)
    h = _gru(bwd_msg(h), h, bw1)
    h = _gru(fwd_msg(bf(h)), h, fw2)
    out2 = snap(h)

    hv = jnp.zeros((16, _H), dtype=jnp.float32)
    hv = _gru(out0, hv, var)
    hv = _gru(out1, hv, var)
    hv = _gru(out2, hv, var)

    hvb = hv.astype(jnp.bfloat16)
    enc = (_dot_nt(hvb[0:_G], wm_ref[0])
           + _dot_nt(hvb[_G:2 * _G], wm_ref[1])
           + _dot_nt(hvb[2 * _G:3 * _G], wm_ref[2])
           + bm_ref[...])
    diff = enc - ke_ref[0]
    del step
    out_ref[0] = jnp.sum(diff * diff, keepdims=True).reshape(1, 1)


def kernel(g_in, g_adj, batch_size, kernel_embeddings, reg_solutions, params):
    B, N, VT = g_in.shape
    Z = kernel_embeddings.shape[1]
    steps = B // _G
    p = params

    wi0 = p["fw"][0]["Wi"].reshape(3, _H, VT).astype(jnp.bfloat16)
    wi5 = (jnp.stack([p["fw"][1]["Wi"], p["fw"][2]["Wi"], p["bw"][0]["Wi"],
                      p["bw"][1]["Wi"], p["var"]["Wi"]])
           .reshape(5, 3, _H, _H).astype(jnp.bfloat16))
    wh6 = (jnp.stack([p["fw"][0]["Wh"], p["fw"][1]["Wh"], p["fw"][2]["Wh"],
                      p["bw"][0]["Wh"], p["bw"][1]["Wh"], p["var"]["Wh"]])
           .reshape(6, 3, _H, _H).astype(jnp.bfloat16))
    bi18 = jnp.stack([p["fw"][0]["bi"], p["fw"][1]["bi"], p["fw"][2]["bi"],
                      p["bw"][0]["bi"], p["bw"][1]["bi"], p["var"]["bi"]]
                     ).reshape(18, _H)
    bh18 = jnp.stack([p["fw"][0]["bh"], p["fw"][1]["bh"], p["fw"][2]["bh"],
                      p["bw"][0]["bh"], p["bw"][1]["bh"], p["var"]["bh"]]
                     ).reshape(18, _H)
    wm3 = p["Wm"].reshape(Z, 3, _H).transpose(1, 0, 2).astype(jnp.bfloat16)
    bm = p["bm"].reshape(1, -1)

    adj4 = g_adj.reshape(steps, _G, N, N)
    gin2 = g_in.reshape(steps, _G * N, VT)
    ke3 = kernel_embeddings.reshape(steps, _G, Z)

    const = lambda shape: pl.BlockSpec(shape, lambda s: (0,) * len(shape))
    in_specs = [
        pl.BlockSpec((1, _G, N, N), lambda s: (s, 0, 0, 0)),
        pl.BlockSpec((1, _G * N, VT), lambda s: (s, 0, 0)),
        pl.BlockSpec((1, _G, Z), lambda s: (s, 0, 0)),
        const(wi0.shape), const(wi5.shape), const(wh6.shape),
        const(bi18.shape), const(bh18.shape), const(wm3.shape),
        const(bm.shape),
    ]

    loss = pl.pallas_call(
        _encode_kernel,
        grid=(steps,),
        in_specs=in_specs,
        out_specs=pl.BlockSpec((1, 1, 1), lambda s: (s, 0, 0)),
        out_shape=jax.ShapeDtypeStruct((steps, 1, 1), jnp.float32),
        compiler_params=pltpu.CompilerParams(
            dimension_semantics=("parallel",),
        ),
    )(adj4, gin2, ke3, wi0, wi5, wh6, bi18, bh18, wm3, bm)
    return jnp.sum(loss)
